# baseline (speedup 1.0000x reference)
import functools

import jax
import jax.numpy as jnp
from jax import lax
from jax.experimental import pallas as pl
from jax.experimental.pallas import tpu as pltpu
from jax.experimental.pallas import tpu_sc as plsc

R = 131072
S = 64
NS = 49
NOUT = 113
HIST_PAD = 0.01
EPS = 1e-05

_info = plsc.get_sparse_core_info()
NCORES = _info.num_cores
NSUB = _info.num_subcores
NWORK = NCORES * NSUB
RAYS_PER_W = R // NWORK
K = 128
NCHUNK = RAYS_PER_W // K
DMA_GROUP = 32


def _body(w_hbm, eb_hbm, nr_hbm, fr_hbm, u_hbm,
          o_se, o_ee, o_sm, o_em,
          w_v, eb_v, nr_v, fr_v, u_v, cdf_v, cdfsh_v, a_v, m_st, e_st,
          sem_i, sem_o):
    wid = lax.axis_index("s") * NCORES + lax.axis_index("c")
    lanes = lax.iota(jnp.int32, 16)
    ones_i = jnp.ones((16,), jnp.int32)
    zeros_i = jnp.zeros((16,), jnp.int32)

    pltpu.sync_copy(u_hbm, u_v)
    u_vec = [u_v[pl.ds(16 * c, 16)] for c in range(4)]
    jvec = [lanes + 16 * c for c in range(4)]
    jmask = [None, None, None, jvec[3] < NS]
    row64 = jnp.full((16,), S, jnp.int32)
    row113 = jnp.full((16,), NOUT, jnp.int32)
    lane0 = lanes < 1

    def do_ray(k):
        row = lax.broadcast(k, (16,))
        w = [plsc.load_gather(w_v, [jvec[c], row]) + HIST_PAD for c in range(4)]
        s_pre = [jnp.sum(w[c]) for c in range(4)]
        total = (s_pre[0] + s_pre[1]) + (s_pre[2] + s_pre[3])
        padding = jnp.maximum(EPS - total, 0.0)
        wadj = padding * (1.0 / S)
        wsum = total + padding
        inv = jnp.ones((16,), jnp.float32) / lax.broadcast(wsum, (16,))
        for c in range(4):
            a_v[pl.ds(16 * c, 16)] = zeros_i
        carry_s = 0.0
        j0sh = []
        for c in range(4):
            wc = w[c] + wadj
            cum = plsc.cumsum(wc) + carry_s
            carry_s = carry_s + s_pre[c] + 16.0 * wadj
            cdfsh_c = jnp.minimum(1.0, (cum - wc) * inv)
            cdfsh_v[pl.ds(16 * c, 16)] = cdfsh_c
            cdf_v[pl.ds(16 * c, 16)] = jnp.minimum(1.0, cum * inv)
            x = jnp.float32(NS) * cdfsh_c - 0.5
            ti = x.astype(jnp.int32)
            j0 = ti + (ti.astype(jnp.float32) < x).astype(jnp.int32)
            j0sh.append(j0)
            plsc.addupdate_scatter(a_v, [j0], ones_i)
        carry_i = jnp.int32(0)
        below = []
        for c in range(4):
            av = a_v[pl.ds(16 * c, 16)]
            bc = plsc.cumsum(av) + carry_i - 1
            carry_i = carry_i + jnp.sum(av)
            below.append(jnp.minimum(bc, S - 1))
        near = plsc.load_gather(nr_v, [row])
        far = plsc.load_gather(fr_v, [row])
        scale = far - near
        for c in range(4):
            b = below[c]
            cdf_g0 = plsc.load_gather(cdfsh_v, [b])
            cdf_g1 = plsc.load_gather(cdf_v, [b])
            bins_g0 = plsc.load_gather(eb_v, [b, row])
            bins_g1 = plsc.load_gather(eb_v, [b + 1, row])
            t = (u_vec[c] - cdf_g0) / (cdf_g1 - cdf_g0)
            t = jnp.clip(t, 0.0, 1.0)
            sv = bins_g0 + t * (bins_g1 - bins_g0)
            sve = near + sv * scale
            rk = jvec[c] + b + 1
            plsc.store_scatter(m_st, [rk, row], sv, mask=jmask[c])
            plsc.store_scatter(e_st, [rk, row], sve, mask=jmask[c])
        for c in range(4):
            rank_eb = jvec[c] + j0sh[c]
            ebv = plsc.load_gather(eb_v, [jvec[c], row])
            ebe = near + ebv * scale
            plsc.store_scatter(m_st, [rank_eb, row], ebv)
            plsc.store_scatter(e_st, [rank_eb, row], ebe)
        eb64 = plsc.load_gather(eb_v, [row64, row])
        plsc.store_scatter(m_st, [row113, row], eb64, mask=lane0)
        plsc.store_scatter(e_st, [row113, row], near + eb64 * scale,
                           mask=lane0)

    def chunk_body(cix, carry):
        base = wid * RAYS_PER_W + cix * K
        handles = []

        def drain():
            for h in handles:
                h.wait()
            handles.clear()

        for i in range(S):
            handles.append(pltpu.async_copy(
                w_hbm.at[pl.ds(i * R + base, K)], w_v.at[i], sem_i))
            if len(handles) >= DMA_GROUP:
                drain()
        drain()
        pltpu.sync_copy(eb_hbm.at[:, pl.ds(base, K)], eb_v)
        pltpu.sync_copy(nr_hbm.at[pl.ds(base, K)], nr_v)
        pltpu.sync_copy(fr_hbm.at[pl.ds(base, K)], fr_v)

        def ray_body(k, rcarry):
            do_ray(k)
            return rcarry

        lax.fori_loop(0, K, ray_body, 0, unroll=False)

        for j in range(NOUT):
            handles.append(pltpu.async_copy(
                e_st.at[j], o_se.at[pl.ds(j * R + base, K)], sem_o))
            handles.append(pltpu.async_copy(
                e_st.at[j + 1], o_ee.at[pl.ds(j * R + base, K)], sem_o))
            handles.append(pltpu.async_copy(
                m_st.at[j], o_sm.at[pl.ds(j * R + base, K)], sem_o))
            handles.append(pltpu.async_copy(
                m_st.at[j + 1], o_em.at[pl.ds(j * R + base, K)], sem_o))
            if len(handles) >= DMA_GROUP:
                drain()
        drain()
        return carry

    lax.fori_loop(0, NCHUNK, chunk_body, 0, unroll=False)


@jax.jit
def _run(wflat, ebt, n1, f1, u):
    f32 = jnp.float32
    mesh = plsc.VectorSubcoreMesh(core_axis_name="c", subcore_axis_name="s")
    out_type = [jax.ShapeDtypeStruct((NOUT * R,), f32) for _ in range(4)]
    scratch = [
        pltpu.VMEM((S, K), f32),
        pltpu.VMEM((S + 1, K), f32),
        pltpu.VMEM((K,), f32),
        pltpu.VMEM((K,), f32),
        pltpu.VMEM((S,), f32),
        pltpu.VMEM((S,), f32),
        pltpu.VMEM((S,), f32),
        pltpu.VMEM((S,), jnp.int32),
        pltpu.VMEM((NOUT + 1, K), f32),
        pltpu.VMEM((NOUT + 1, K), f32),
        pltpu.SemaphoreType.DMA,
        pltpu.SemaphoreType.DMA,
    ]
    kfn = functools.partial(
        pl.kernel, mesh=mesh, out_type=out_type, scratch_types=scratch,
        compiler_params=pltpu.CompilerParams(
            needs_layout_passes=False, use_tc_tiling_on_sc=False),
    )(_body)
    return kfn(wflat, ebt, n1, f1, u)


def kernel(weights, existing_bins, nears, fars):
    wflat = jnp.reshape(weights[..., 0].T, (S * R,))
    ebt = existing_bins.T
    n1 = nears[:, 0]
    f1 = fars[:, 0]
    u = jnp.linspace(0.0, 1.0 - 1.0 / NS, NS, dtype=jnp.float32) + jnp.float32(
        1.0 / (2 * NS))
    u = jnp.concatenate([u, jnp.zeros((S - NS,), jnp.float32)])
    se, ee, sm, em = _run(wflat, ebt, n1, f1, u)
    out = tuple(
        jnp.reshape(x, (NOUT, R)).T[..., None] for x in (se, ee, sm, em))
    return out
